# Initial kernel scaffold; baseline (speedup 1.0000x reference)
import jax
import jax.numpy as jnp
from jax.experimental import pallas as pl


def kernel(x, pe):
    raise NotImplementedError("write your pallas kernel here")


# speedup vs baseline: 2.4703x; 2.4703x over previous
import functools

import jax
import jax.numpy as jnp
from jax import lax
from jax.experimental import pallas as pl
from jax.experimental.pallas import tpu as pltpu
from jax.experimental.pallas import tpu_sc as plsc

D_MODEL = 768
SEQ_LEN = 8192
BATCH = 4

_info = plsc.get_sparse_core_info()
_NC = _info.num_cores
_NS = _info.num_subcores
_NW = _NC * _NS
_B_TOTAL = BATCH * SEQ_LEN
_B_PER_W = _B_TOTAL // _NW
_CHUNK = 64
_N_CHUNKS = _B_PER_W // _CHUNK

_mesh = plsc.VectorSubcoreMesh(core_axis_name="c", subcore_axis_name="s")


@functools.partial(
    pl.kernel,
    mesh=_mesh,
    out_type=jax.ShapeDtypeStruct((_B_TOTAL, D_MODEL), jnp.float32),
    scratch_types=[
        pltpu.VMEM((_N_CHUNKS, _CHUNK), jnp.int32),
        pltpu.VMEM((2, _CHUNK, D_MODEL), jnp.float32),
        pltpu.SemaphoreType.DMA,
        pltpu.SemaphoreType.DMA,
    ],
)
def _gather_kernel(idx_hbm, table_hbm, out_hbm, idx_v, rows_v, sem0, sem1):
    wid = lax.axis_index("s") * _NC + lax.axis_index("c")
    base = wid * _B_PER_W
    pltpu.sync_copy(idx_hbm.at[wid], idx_v)
    sems = (sem0, sem1)
    copies = [None, None]
    copies[0] = pltpu.async_copy(table_hbm.at[idx_v.at[0]], rows_v.at[0], sem0)
    for j in range(_N_CHUNKS):
        nb = (j + 1) % 2
        if j + 1 < _N_CHUNKS:
            copies[nb] = pltpu.async_copy(
                table_hbm.at[idx_v.at[j + 1]], rows_v.at[nb], sems[nb])
        copies[j % 2].wait()
        pltpu.sync_copy(
            rows_v.at[j % 2], out_hbm.at[pl.ds(base + j * _CHUNK, _CHUNK)])


def kernel(x, pe):
    idx = x.reshape(_NW, _N_CHUNKS, _CHUNK)
    out = _gather_kernel(idx, pe)
    return out.reshape(BATCH, SEQ_LEN, D_MODEL)


# speedup vs baseline: 2.4734x; 1.0013x over previous
import functools

import jax
import jax.numpy as jnp
from jax import lax
from jax.experimental import pallas as pl
from jax.experimental.pallas import tpu as pltpu
from jax.experimental.pallas import tpu_sc as plsc

D_MODEL = 768
SEQ_LEN = 8192
BATCH = 4

_info = plsc.get_sparse_core_info()
_NC = _info.num_cores
_NS = _info.num_subcores
_NW = _NC * _NS
_B_TOTAL = BATCH * SEQ_LEN
_B_PER_W = _B_TOTAL // _NW
_CHUNK = 64
_N_CHUNKS = _B_PER_W // _CHUNK

_mesh = plsc.VectorSubcoreMesh(core_axis_name="c", subcore_axis_name="s")


@functools.partial(
    pl.kernel,
    mesh=_mesh,
    out_type=jax.ShapeDtypeStruct((_B_TOTAL, D_MODEL), jnp.float32),
    scratch_types=[
        pltpu.VMEM((_N_CHUNKS, _CHUNK), jnp.int32),
        pltpu.VMEM((2, _CHUNK, D_MODEL), jnp.float32),
        pltpu.SemaphoreType.DMA,
        pltpu.SemaphoreType.DMA,
        pltpu.SemaphoreType.DMA,
        pltpu.SemaphoreType.DMA,
    ],
)
def _gather_kernel(idx_hbm, table_hbm, out_hbm, idx_v, rows_v,
                   gsem0, gsem1, ssem0, ssem1):
    wid = lax.axis_index("s") * _NC + lax.axis_index("c")
    base = wid * _B_PER_W
    pltpu.sync_copy(idx_hbm.at[wid], idx_v)
    gsems = (gsem0, gsem1)
    ssems = (ssem0, ssem1)
    gathers = [None, None]
    scatters = [None, None]
    gathers[0] = pltpu.async_copy(
        table_hbm.at[idx_v.at[0]], rows_v.at[0], gsem0)
    for j in range(_N_CHUNKS):
        cb = j % 2
        nb = (j + 1) % 2
        if j + 1 < _N_CHUNKS:
            if scatters[nb] is not None:
                scatters[nb].wait()
            gathers[nb] = pltpu.async_copy(
                table_hbm.at[idx_v.at[j + 1]], rows_v.at[nb], gsems[nb])
        gathers[cb].wait()
        scatters[cb] = pltpu.async_copy(
            rows_v.at[cb], out_hbm.at[pl.ds(base + j * _CHUNK, _CHUNK)],
            ssems[cb])
    scatters[0].wait()
    scatters[1].wait()


def kernel(x, pe):
    idx = x.reshape(_NW, _N_CHUNKS, _CHUNK)
    out = _gather_kernel(idx, pe)
    return out.reshape(BATCH, SEQ_LEN, D_MODEL)
